# baseline (speedup 1.0000x reference)
import functools

import jax
import jax.numpy as jnp
from jax import lax
from jax.experimental import pallas as pl
from jax.experimental.pallas import tpu as pltpu
from jax.experimental.pallas import tpu_sc as plsc

NC = 2
NS = 16
NW = NC * NS


def _make_sc_gather(V, D, sizes):
    bs = [B // NW for B in sizes]
    ch = [min(b, 128) for b in bs]
    nc = [b // c for b, c in zip(bs, ch)]
    mesh = plsc.VectorSubcoreMesh(core_axis_name="c", subcore_axis_name="s")
    scratch = []
    for b, c, n in zip(bs, ch, nc):
        scratch += [pltpu.VMEM((n, c), jnp.int32),
                    pltpu.VMEM((b, D), jnp.float32)]
    scratch += [pltpu.SemaphoreType.DMA, pltpu.SemaphoreType.DMA]
    G = len(sizes)

    @functools.partial(
        pl.kernel,
        out_type=tuple(jax.ShapeDtypeStruct((B, D), jnp.float32)
                       for B in sizes),
        mesh=mesh,
        scratch_types=scratch,
    )
    def gather(table, *refs):
        idxs = refs[:G]
        outs = refs[G:2 * G]
        scr = refs[2 * G:]
        gsem, wsem = scr[-2], scr[-1]
        wid = lax.axis_index("s") * NC + lax.axis_index("c")
        fired = []
        for g in range(G):
            iv, rv = scr[2 * g], scr[2 * g + 1]
            base = wid * bs[g]
            for j in range(nc[g]):
                pltpu.sync_copy(idxs[g].at[pl.ds(base + j * ch[g], ch[g])],
                                iv.at[j])
            for j in range(nc[g]):
                fired.append((pltpu.async_copy(
                    table.at[iv.at[j]],
                    rv.at[pl.ds(j * ch[g], ch[g])], gsem), g, j))
        wbs = []
        for cp, g, j in fired:
            cp.wait()
            rv = scr[2 * g + 1]
            base = wid * bs[g]
            wbs.append(pltpu.async_copy(
                rv.at[pl.ds(j * ch[g], ch[g])],
                outs[g].at[pl.ds(base + j * ch[g], ch[g])], wsem))
        for cp in wbs:
            cp.wait()

    return gather


def _mm_part(dif, g, kblk_off, p, tail, bk):
    M = dif.shape[0]
    Ks, D = g.shape
    nk = Ks // bk

    dif_spec = pl.BlockSpec((M, bk), lambda k: (0, k + kblk_off))
    g_spec = pl.BlockSpec((bk, D), lambda k: (k, 0))
    full = pl.BlockSpec((M, D), lambda k: (0, 0))
    wspec = pl.BlockSpec((D, D), lambda k: (0, 0))

    if tail is None:
        def body(dif_ref, g_ref, *rest):
            (p_ref, out_ref) = ((rest[0], rest[1]) if p is not None
                                else (None, rest[0]))
            k = pl.program_id(0)
            contrib = jnp.dot(dif_ref[...], g_ref[...],
                              preferred_element_type=jnp.float32)

            @pl.when(k == 0)
            def _():
                out_ref[...] = (contrib if p_ref is None
                                else p_ref[...] + contrib)

            @pl.when(k > 0)
            def _():
                out_ref[...] += contrib

        in_specs = [dif_spec, g_spec] + ([full] if p is not None else [])
        args = (dif, g) + ((p,) if p is not None else ())
        return pl.pallas_call(
            body,
            grid=(nk,),
            in_specs=in_specs,
            out_specs=full,
            out_shape=jax.ShapeDtypeStruct((M, D), jnp.float32),
        )(*args)

    d2, wa, wb = tail

    def body(dif_ref, g_ref, *rest):
        if p is not None:
            p_ref, d2_ref, wa_ref, wb_ref, out_ref, acc_ref = rest
        else:
            d2_ref, wa_ref, wb_ref, out_ref, acc_ref = rest
            p_ref = None
        k = pl.program_id(0)
        contrib = jnp.dot(dif_ref[...], g_ref[...],
                          preferred_element_type=jnp.float32)

        @pl.when(k == 0)
        def _():
            acc_ref[...] = (contrib if p_ref is None
                            else p_ref[...] + contrib)

        @pl.when(k > 0)
        def _():
            acc_ref[...] += contrib

        @pl.when(k == nk - 1)
        def _():
            out_ref[...] = jnp.maximum(
                jnp.dot(acc_ref[...], wa_ref[...],
                        preferred_element_type=jnp.float32)
                + jnp.dot(d2_ref[...], wb_ref[...],
                          preferred_element_type=jnp.float32),
                0.0)

    in_specs = ([dif_spec, g_spec]
                + ([full] if p is not None else [])
                + [full, wspec, wspec])
    args = (dif, g) + ((p,) if p is not None else ()) + (d2, wa, wb)
    return pl.pallas_call(
        body,
        grid=(nk,),
        in_specs=in_specs,
        out_specs=full,
        out_shape=jax.ShapeDtypeStruct((M, D), jnp.float32),
        scratch_shapes=[pltpu.VMEM((M, D), jnp.float32)],
    )(*args)


def _mm_small_onehot(dif, x, idx_s, idx_d, wa, wb):
    M, Kv = dif.shape
    V2, D = x.shape
    bv = 512
    nv = V2 // bv

    def body(dif_ref, x_ref, is_ref, id_ref, wa_ref, wb_ref, out_ref,
             g1_ref, d1_ref):
        c = pl.program_id(0)
        base = c * bv
        ids_s = base + jax.lax.broadcasted_iota(jnp.int32, (Kv, bv), 1)
        o_s = (ids_s == is_ref[...]).astype(jnp.float32)
        cs = jnp.dot(o_s, x_ref[...], preferred_element_type=jnp.float32)
        ids_d = base + jax.lax.broadcasted_iota(jnp.int32, (M, bv), 1)
        o_d = (ids_d == id_ref[...]).astype(jnp.float32)
        cd = jnp.dot(o_d, x_ref[...], preferred_element_type=jnp.float32)

        @pl.when(c == 0)
        def _():
            g1_ref[...] = cs
            d1_ref[...] = cd

        @pl.when(c > 0)
        def _():
            g1_ref[...] += cs
            d1_ref[...] += cd

        @pl.when(c == nv - 1)
        def _():
            agg = jnp.dot(dif_ref[...], g1_ref[...],
                          preferred_element_type=jnp.float32)
            out_ref[...] = jnp.maximum(
                jnp.dot(agg, wa_ref[...], preferred_element_type=jnp.float32)
                + jnp.dot(d1_ref[...], wb_ref[...],
                          preferred_element_type=jnp.float32),
                0.0)

    return pl.pallas_call(
        body,
        grid=(nv,),
        in_specs=[
            pl.BlockSpec((M, Kv), lambda c: (0, 0)),
            pl.BlockSpec((bv, D), lambda c: (c, 0)),
            pl.BlockSpec((Kv, 1), lambda c: (0, 0)),
            pl.BlockSpec((M, 1), lambda c: (0, 0)),
            pl.BlockSpec((D, D), lambda c: (0, 0)),
            pl.BlockSpec((D, D), lambda c: (0, 0)),
        ],
        out_specs=pl.BlockSpec((M, D), lambda c: (0, 0)),
        out_shape=jax.ShapeDtypeStruct((M, D), jnp.float32),
        scratch_shapes=[pltpu.VMEM((Kv, D), jnp.float32),
                        pltpu.VMEM((M, D), jnp.float32)],
    )(dif, x, idx_s.reshape(Kv, 1), idx_d.reshape(M, 1), wa, wb)


BK = 1024


def kernel(src_nodes, dstsrc2src_1, dstsrc2src_2, dstsrc2dst_1, dstsrc2dst_2,
           dif_mat_1, dif_mat_2, w1, w2):
    V, D = src_nodes.shape
    w1a, w1b = w1[:D], w1[D:]
    w2a, w2b = w2[:D], w2[D:]

    gather1 = _make_sc_gather(V, D,
                              (dstsrc2src_2.shape[0], dstsrc2dst_2.shape[0]))
    g2, d2 = gather1(src_nodes, dstsrc2src_2, dstsrc2dst_2)
    x = _mm_part(dif_mat_2, g2, 0, None, (d2, w1a, w1b), BK)

    return _mm_small_onehot(dif_mat_1, x, dstsrc2src_1, dstsrc2dst_1,
                            w2a, w2b)


# speedup vs baseline: 1.0032x; 1.0032x over previous
import functools

import jax
import jax.numpy as jnp
from jax import lax
from jax.experimental import pallas as pl
from jax.experimental.pallas import tpu as pltpu
from jax.experimental.pallas import tpu_sc as plsc

NC = 2
NS = 16
NW = NC * NS


def _make_sc_gather(V, D, sizes):
    bs = [B // NW for B in sizes]
    ch = [min(b, 128) for b in bs]
    nc = [b // c for b, c in zip(bs, ch)]
    mesh = plsc.VectorSubcoreMesh(core_axis_name="c", subcore_axis_name="s")
    scratch = []
    for b, c, n in zip(bs, ch, nc):
        scratch += [pltpu.VMEM((n, c), jnp.int32),
                    pltpu.VMEM((b, D), jnp.float32)]
    scratch += [pltpu.SemaphoreType.DMA, pltpu.SemaphoreType.DMA]
    G = len(sizes)

    @functools.partial(
        pl.kernel,
        out_type=tuple(jax.ShapeDtypeStruct((B, D), jnp.float32)
                       for B in sizes),
        mesh=mesh,
        scratch_types=scratch,
    )
    def gather(table, *refs):
        idxs = refs[:G]
        outs = refs[G:2 * G]
        scr = refs[2 * G:]
        gsem, wsem = scr[-2], scr[-1]
        wid = lax.axis_index("s") * NC + lax.axis_index("c")
        fired = []
        for g in range(G):
            iv, rv = scr[2 * g], scr[2 * g + 1]
            base = wid * bs[g]
            for j in range(nc[g]):
                pltpu.sync_copy(idxs[g].at[pl.ds(base + j * ch[g], ch[g])],
                                iv.at[j])
            for j in range(nc[g]):
                fired.append((pltpu.async_copy(
                    table.at[iv.at[j]],
                    rv.at[pl.ds(j * ch[g], ch[g])], gsem), g, j))
        wbs = []
        for cp, g, j in fired:
            cp.wait()
            rv = scr[2 * g + 1]
            base = wid * bs[g]
            wbs.append(pltpu.async_copy(
                rv.at[pl.ds(j * ch[g], ch[g])],
                outs[g].at[pl.ds(base + j * ch[g], ch[g])], wsem))
        for cp in wbs:
            cp.wait()

    return gather


def _mm_part(dif, g, kblk_off, p, tail, bk):
    M = dif.shape[0]
    Ks, D = g.shape
    nk = Ks // bk

    dif_spec = pl.BlockSpec((M, bk), lambda k: (0, k + kblk_off))
    g_spec = pl.BlockSpec((bk, D), lambda k: (k, 0))
    full = pl.BlockSpec((M, D), lambda k: (0, 0))
    wspec = pl.BlockSpec((D, D), lambda k: (0, 0))

    if tail is None:
        def body(dif_ref, g_ref, *rest):
            (p_ref, out_ref) = ((rest[0], rest[1]) if p is not None
                                else (None, rest[0]))
            k = pl.program_id(0)
            contrib = jnp.dot(dif_ref[...], g_ref[...],
                              preferred_element_type=jnp.float32)

            @pl.when(k == 0)
            def _():
                out_ref[...] = (contrib if p_ref is None
                                else p_ref[...] + contrib)

            @pl.when(k > 0)
            def _():
                out_ref[...] += contrib

        in_specs = [dif_spec, g_spec] + ([full] if p is not None else [])
        args = (dif, g) + ((p,) if p is not None else ())
        return pl.pallas_call(
            body,
            grid=(nk,),
            in_specs=in_specs,
            out_specs=full,
            out_shape=jax.ShapeDtypeStruct((M, D), jnp.float32),
        )(*args)

    d2, wa, wb = tail

    def body(dif_ref, g_ref, *rest):
        if p is not None:
            p_ref, d2_ref, wa_ref, wb_ref, out_ref, acc_ref = rest
        else:
            d2_ref, wa_ref, wb_ref, out_ref, acc_ref = rest
            p_ref = None
        k = pl.program_id(0)
        contrib = jnp.dot(dif_ref[...], g_ref[...],
                          preferred_element_type=jnp.float32)

        @pl.when(k == 0)
        def _():
            acc_ref[...] = (contrib if p_ref is None
                            else p_ref[...] + contrib)

        @pl.when(k > 0)
        def _():
            acc_ref[...] += contrib

        @pl.when(k == nk - 1)
        def _():
            out_ref[...] = jnp.maximum(
                jnp.dot(acc_ref[...], wa_ref[...],
                        preferred_element_type=jnp.float32)
                + jnp.dot(d2_ref[...], wb_ref[...],
                          preferred_element_type=jnp.float32),
                0.0)

    in_specs = ([dif_spec, g_spec]
                + ([full] if p is not None else [])
                + [full, wspec, wspec])
    args = (dif, g) + ((p,) if p is not None else ()) + (d2, wa, wb)
    return pl.pallas_call(
        body,
        grid=(nk,),
        in_specs=in_specs,
        out_specs=full,
        out_shape=jax.ShapeDtypeStruct((M, D), jnp.float32),
        scratch_shapes=[pltpu.VMEM((M, D), jnp.float32)],
    )(*args)


def _layer2_onehot(dif, p, d2, w1a, w1b, idx_s, idx_d, wa, wb):
    M, Kv = dif.shape
    V2, D = p.shape
    bv = 512
    nv = V2 // bv

    def body(dif_ref, p_ref, d2_ref, w1a_ref, w1b_ref, is_ref, id_ref,
             wa_ref, wb_ref, out_ref, x_ref, g1_ref, d1_ref):
        c = pl.program_id(0)

        @pl.when(c == 0)
        def _():
            x_ref[...] = jnp.maximum(
                jnp.dot(p_ref[...], w1a_ref[...],
                        preferred_element_type=jnp.float32)
                + jnp.dot(d2_ref[...], w1b_ref[...],
                          preferred_element_type=jnp.float32),
                0.0)

        base = c * bv
        xc = x_ref[pl.ds(base, bv), :]
        ids_s = base + jax.lax.broadcasted_iota(jnp.int32, (Kv, bv), 1)
        o_s = (ids_s == is_ref[...]).astype(jnp.float32)
        cs = jnp.dot(o_s, xc, preferred_element_type=jnp.float32)
        ids_d = base + jax.lax.broadcasted_iota(jnp.int32, (M, bv), 1)
        o_d = (ids_d == id_ref[...]).astype(jnp.float32)
        cd = jnp.dot(o_d, xc, preferred_element_type=jnp.float32)

        @pl.when(c == 0)
        def _():
            g1_ref[...] = cs
            d1_ref[...] = cd

        @pl.when(c > 0)
        def _():
            g1_ref[...] += cs
            d1_ref[...] += cd

        @pl.when(c == nv - 1)
        def _():
            agg = jnp.dot(dif_ref[...], g1_ref[...],
                          preferred_element_type=jnp.float32)
            out_ref[...] = jnp.maximum(
                jnp.dot(agg, wa_ref[...], preferred_element_type=jnp.float32)
                + jnp.dot(d1_ref[...], wb_ref[...],
                          preferred_element_type=jnp.float32),
                0.0)

    full = lambda c: (0, 0)
    return pl.pallas_call(
        body,
        grid=(nv,),
        in_specs=[
            pl.BlockSpec((M, Kv), full),
            pl.BlockSpec((V2, D), full),
            pl.BlockSpec((V2, D), full),
            pl.BlockSpec((D, D), full),
            pl.BlockSpec((D, D), full),
            pl.BlockSpec((Kv, 1), full),
            pl.BlockSpec((M, 1), full),
            pl.BlockSpec((D, D), full),
            pl.BlockSpec((D, D), full),
        ],
        out_specs=pl.BlockSpec((M, D), full),
        out_shape=jax.ShapeDtypeStruct((M, D), jnp.float32),
        scratch_shapes=[pltpu.VMEM((V2, D), jnp.float32),
                        pltpu.VMEM((Kv, D), jnp.float32),
                        pltpu.VMEM((M, D), jnp.float32)],
    )(dif, p, d2, w1a, w1b,
      idx_s.reshape(Kv, 1), idx_d.reshape(M, 1), wa, wb)


BK = 1024


def kernel(src_nodes, dstsrc2src_1, dstsrc2src_2, dstsrc2dst_1, dstsrc2dst_2,
           dif_mat_1, dif_mat_2, w1, w2):
    V, D = src_nodes.shape
    w1a, w1b = w1[:D], w1[D:]
    w2a, w2b = w2[:D], w2[D:]

    gather_src = _make_sc_gather(V, D, (dstsrc2src_2.shape[0],))
    (g2,) = gather_src(src_nodes, dstsrc2src_2)
    gather_dst = _make_sc_gather(V, D, (dstsrc2dst_2.shape[0],))
    (d2,) = gather_dst(src_nodes, dstsrc2dst_2)
    p = _mm_part(dif_mat_2, g2, 0, None, None, BK)

    return _layer2_onehot(dif_mat_1, p, d2, w1a, w1b,
                          dstsrc2src_1, dstsrc2dst_1, w2a, w2b)
